# baseline (speedup 1.0000x reference)
import functools

import jax
import jax.numpy as jnp
from jax import lax
from jax.experimental import pallas as pl
from jax.experimental.pallas import tpu as pltpu
from jax.experimental.pallas import tpu_sc as plsc

R = 16384
C = 1000
K = int(R * 0.7)

NC = 2
NS = 16
NW = NC * NS
L = 16

R_SC = 8192
R_TC = R - R_SC
RPW = R_SC // NW
GR = 32
NG = RPW // GR

BR = 2048
NB_TC = R_TC // BR

_UNROLL = 25


def _sc_body(pred_hbm, tgt_hbm, m_hbm, s_hbm, psc_hbm,
             xbuf0, xbuf1, tvall, macc, sacc, pacc, sem0, sem1):
    wid = lax.axis_index("s") * NC + lax.axis_index("c")
    base = wid * RPW
    lane = lax.broadcasted_iota(jnp.int32, (L,), 0)
    lanec = lane * C

    pltpu.sync_copy(tgt_hbm.at[pl.ds(base, RPW)], tvall)

    def start(g, buf, sem):
        return pltpu.async_copy(
            pred_hbm.at[pl.ds(base + g * GR, GR)], buf, sem
        )

    def compute(g, buf):
        for h in range(GR // L):
            rowv = lane + h * L

            def p1(jo, m):
                for u in range(_UNROLL):
                    j = jo * _UNROLL + u
                    col = j + jnp.zeros((L,), jnp.int32)
                    v = plsc.load_gather(buf, [rowv, col])
                    m = jnp.maximum(m, v)
                return m

            m = lax.fori_loop(0, C // _UNROLL, p1,
                              jnp.full((L,), -jnp.inf, jnp.float32))

            def p2(jo, s):
                for u in range(_UNROLL):
                    j = jo * _UNROLL + u
                    col = j + jnp.zeros((L,), jnp.int32)
                    v = plsc.load_gather(buf, [rowv, col])
                    s = s + jnp.exp(v - m)
                return s

            s = lax.fori_loop(0, C // _UNROLL, p2,
                              jnp.zeros((L,), jnp.float32))

            r0 = g * GR + h * L
            tv = tvall[pl.ds(r0, L)]
            pk = plsc.load_gather(buf, [rowv, tv])
            macc[pl.ds(r0, L)] = m
            sacc[pl.ds(r0, L)] = s
            pacc[pl.ds(r0, L)] = pk

    start(0, xbuf0, sem0)
    start(1, xbuf1, sem1)

    def wait(g, buf, sem):
        pltpu.make_async_copy(
            pred_hbm.at[pl.ds(base + g * GR, GR)], buf, sem
        ).wait()

    def outer(gg, _):
        g0 = gg * 2
        wait(g0, xbuf0, sem0)
        compute(g0, xbuf0)

        @pl.when(g0 + 2 < NG)
        def _():
            start(g0 + 2, xbuf0, sem0)

        wait(g0 + 1, xbuf1, sem1)
        compute(g0 + 1, xbuf1)

        @pl.when(g0 + 3 < NG)
        def _():
            start(g0 + 3, xbuf1, sem1)

        return 0

    lax.fori_loop(0, NG // 2, outer, 0)

    pltpu.sync_copy(macc, m_hbm.at[pl.ds(base, RPW)])
    pltpu.sync_copy(sacc, s_hbm.at[pl.ds(base, RPW)])
    pltpu.sync_copy(pacc, psc_hbm.at[pl.ds(base, RPW)])


_sc_ce = functools.partial(
    pl.kernel,
    out_type=[
        jax.ShapeDtypeStruct((R_SC,), jnp.float32),
        jax.ShapeDtypeStruct((R_SC,), jnp.float32),
        jax.ShapeDtypeStruct((R_SC,), jnp.float32),
    ],
    mesh=plsc.VectorSubcoreMesh(core_axis_name="c", subcore_axis_name="s"),
    compiler_params=pltpu.CompilerParams(needs_layout_passes=False),
    scratch_types=[
        pltpu.VMEM((GR, C), jnp.float32),
        pltpu.VMEM((GR, C), jnp.float32),
        pltpu.VMEM((RPW,), jnp.int32),
        pltpu.VMEM((RPW,), jnp.float32),
        pltpu.VMEM((RPW,), jnp.float32),
        pltpu.VMEM((RPW,), jnp.float32),
        pltpu.SemaphoreType.DMA,
        pltpu.SemaphoreType.DMA,
    ],
)(_sc_body)


def _tc_loss_body(pred_ref, tgt_ref, out_ref):
    x = pred_ref[...]
    m = jnp.max(x, axis=1)
    s = jnp.sum(jnp.exp(x - m[:, None]), axis=1)
    tgt = tgt_ref[0, 0, :]
    col = lax.broadcasted_iota(jnp.int32, (BR, C), 1)
    picked = jnp.sum(jnp.where(col == tgt[:, None], x, 0.0), axis=1)
    out_ref[0, 0, :] = m + jnp.log(s) - picked


def _tc_loss(pred, tgt):
    tgt3 = tgt[R_SC:].reshape(NB_TC, 1, BR)
    out = pl.pallas_call(
        _tc_loss_body,
        grid=(NB_TC,),
        in_specs=[
            pl.BlockSpec((BR, C), lambda i: (i + R_SC // BR, 0)),
            pl.BlockSpec((1, 1, BR), lambda i: (i, 0, 0)),
        ],
        out_specs=pl.BlockSpec((1, 1, BR), lambda i: (i, 0, 0)),
        out_shape=jax.ShapeDtypeStruct((NB_TC, 1, BR), jnp.float32),
    )(pred, tgt3)
    return out.reshape(R_TC)


def _select_body(m_ref, s_ref, psc_ref, ltc_ref, out_ref):
    loss_sc = m_ref[...] + jnp.log(s_ref[...]) - psc_ref[...]
    vals = jnp.concatenate([loss_sc, ltc_ref[...]])
    u = lax.bitcast_convert_type(vals, jnp.uint32)
    sk = u ^ jnp.where(
        u >= jnp.uint32(0x80000000),
        jnp.uint32(0xFFFFFFFF),
        jnp.uint32(0x80000000),
    )

    def body(it, p):
        cand = p | (jnp.uint32(1) << (31 - it).astype(jnp.uint32))
        cnt = jnp.sum((sk >= cand).astype(jnp.int32))
        return jnp.where(cnt >= K, cand, p)

    p = lax.fori_loop(0, 32, body, jnp.uint32(0))

    gt = sk > p
    cnt_gt = jnp.sum(gt.astype(jnp.int32))
    sum_gt = jnp.sum(jnp.where(gt, vals, 0.0))
    orig = jnp.where(
        (p & jnp.uint32(0x80000000)) != jnp.uint32(0),
        p ^ jnp.uint32(0x80000000),
        ~p,
    )
    tau = lax.bitcast_convert_type(orig, jnp.float32)
    total = sum_gt + (K - cnt_gt).astype(jnp.float32) * tau
    out_ref[0, 0] = total / K


def _select(m_sc, s_sc, p_sc, loss_tc):
    return pl.pallas_call(
        _select_body,
        out_specs=pl.BlockSpec(memory_space=pltpu.SMEM),
        out_shape=jax.ShapeDtypeStruct((1, 1), jnp.float32),
    )(m_sc, s_sc, p_sc, loss_tc)


def kernel(pred, target):
    tgt = target.astype(jnp.int32)
    m_sc, s_sc, p_sc = _sc_ce(pred, tgt)
    loss_tc = _tc_loss(pred, tgt)
    out = _select(m_sc, s_sc, p_sc, loss_tc)
    return out[0, 0]


# speedup vs baseline: 9.8567x; 9.8567x over previous
import jax
import jax.numpy as jnp
from jax import lax
from jax.experimental import pallas as pl
from jax.experimental.pallas import tpu as pltpu

R = 16384
C = 1000
K = int(R * 0.7)
BR = 2048
NB = R // BR


def _ohem_kernel(predt_ref, tgt_ref, out_ref, loss_sc):
    i = pl.program_id(0)
    x = predt_ref[...]
    m = jnp.max(x, axis=0)
    s = jnp.sum(jnp.exp(x - m[None, :]), axis=0)
    lse = m + jnp.log(s)
    tgt = tgt_ref[0, 0, :]
    row = lax.broadcasted_iota(jnp.int32, (C, BR), 0)
    picked = jnp.sum(jnp.where(row == tgt[None, :], x, 0.0), axis=0)
    loss_sc[i, :] = lse - picked

    @pl.when(i == NB - 1)
    def _():
        vals = loss_sc[...]
        u = lax.bitcast_convert_type(vals, jnp.uint32)
        sk = u ^ jnp.where(
            u >= jnp.uint32(0x80000000),
            jnp.uint32(0xFFFFFFFF),
            jnp.uint32(0x80000000),
        )

        def body(it, p):
            cand = p | (jnp.uint32(1) << (31 - it).astype(jnp.uint32))
            cnt = jnp.sum((sk >= cand).astype(jnp.int32))
            return jnp.where(cnt >= K, cand, p)

        p = lax.fori_loop(0, 32, body, jnp.uint32(0))

        gt = sk > p
        cnt_gt = jnp.sum(gt.astype(jnp.int32))
        sum_gt = jnp.sum(jnp.where(gt, vals, 0.0))
        orig = jnp.where(
            (p & jnp.uint32(0x80000000)) != jnp.uint32(0),
            p ^ jnp.uint32(0x80000000),
            ~p,
        )
        tau = lax.bitcast_convert_type(orig, jnp.float32)
        total = sum_gt + (K - cnt_gt).astype(jnp.float32) * tau
        out_ref[0, 0] = total / K


def kernel(pred, target):
    predt = pred.T
    tgt = target.astype(jnp.int32).reshape(NB, 1, BR)
    out = pl.pallas_call(
        _ohem_kernel,
        grid=(NB,),
        in_specs=[
            pl.BlockSpec((C, BR), lambda i: (0, i)),
            pl.BlockSpec((1, 1, BR), lambda i: (i, 0, 0)),
        ],
        out_specs=pl.BlockSpec(
            (1, 1), lambda i: (0, 0), memory_space=pltpu.SMEM
        ),
        out_shape=jax.ShapeDtypeStruct((1, 1), jnp.float32),
        scratch_shapes=[pltpu.VMEM((NB, BR), jnp.float32)],
    )(predt, tgt)
    return out[0, 0]


# speedup vs baseline: 10.1448x; 1.0292x over previous
import jax
import jax.numpy as jnp
from jax import lax
from jax.experimental import pallas as pl
from jax.experimental.pallas import tpu as pltpu

R = 16384
C = 1000
K = int(R * 0.7)
BR = 2048
NB = R // BR


def _ohem_kernel(predt_ref, tgt_ref, out_ref, loss_sc):
    i = pl.program_id(0)
    x = predt_ref[...]
    m = jnp.max(x, axis=0)
    e = jnp.exp(x - m[None, :])
    s = lax.dot_general(
        jnp.ones((1, C), jnp.float32), e,
        (((1,), (0,)), ((), ())),
        preferred_element_type=jnp.float32,
    )[0]
    lse = m + jnp.log(s)
    tgt = tgt_ref[0, 0, :]
    row = lax.broadcasted_iota(jnp.int32, (C, BR), 0)
    picked = jnp.sum(jnp.where(row == tgt[None, :], x, 0.0), axis=0)
    loss_sc[i, :] = lse - picked

    @pl.when(i == NB - 1)
    def _():
        vals = loss_sc[...]
        u = lax.bitcast_convert_type(vals, jnp.uint32)
        sk = u ^ jnp.where(
            u >= jnp.uint32(0x80000000),
            jnp.uint32(0xFFFFFFFF),
            jnp.uint32(0x80000000),
        )

        def body(it, p):
            cand = p | (jnp.uint32(1) << (31 - it).astype(jnp.uint32))
            cnt = jnp.sum((sk >= cand).astype(jnp.int32))
            return jnp.where(cnt >= K, cand, p)

        p = lax.fori_loop(0, 32, body, jnp.uint32(0))

        gt = sk > p
        cnt_gt = jnp.sum(gt.astype(jnp.int32))
        sum_gt = jnp.sum(jnp.where(gt, vals, 0.0))
        orig = jnp.where(
            (p & jnp.uint32(0x80000000)) != jnp.uint32(0),
            p ^ jnp.uint32(0x80000000),
            ~p,
        )
        tau = lax.bitcast_convert_type(orig, jnp.float32)
        total = sum_gt + (K - cnt_gt).astype(jnp.float32) * tau
        out_ref[0, 0] = total / K


def kernel(pred, target):
    predt = pred.T
    tgt = target.astype(jnp.int32).reshape(NB, 1, BR)
    out = pl.pallas_call(
        _ohem_kernel,
        grid=(NB,),
        in_specs=[
            pl.BlockSpec((C, BR), lambda i: (0, i)),
            pl.BlockSpec((1, 1, BR), lambda i: (i, 0, 0)),
        ],
        out_specs=pl.BlockSpec(
            (1, 1), lambda i: (0, 0), memory_space=pltpu.SMEM
        ),
        out_shape=jax.ShapeDtypeStruct((1, 1), jnp.float32),
        scratch_shapes=[pltpu.VMEM((NB, BR), jnp.float32)],
    )(predt, tgt)
    return out[0, 0]
